# Initial kernel scaffold; baseline (speedup 1.0000x reference)
import jax
import jax.numpy as jnp
from jax.experimental import pallas as pl


def kernel(xyz, feat_in, Wf, bf, gf, betaf, Wa, ba, ga, betaa):
    raise NotImplementedError("write your pallas kernel here")


# speedup vs baseline: 1.0009x; 1.0009x over previous
import functools

import jax
import jax.numpy as jnp
import numpy as np
from jax.experimental import pallas as pl
from jax.experimental.pallas import tpu as pltpu

B, N, C_IN, M, K, OUT = 8, 8192, 64, 2048, 32, 128
D_CAT = 3 + C_IN
D_PAD = 80
EPS = 1e-5

R_BLK = 256


def _mlp_body(g_ref, wf_ref, bf_ref, wac_ref, waf_ref, ba_ref, out_ref):
    x = g_ref[...]
    fp = jnp.dot(x, wf_ref[...], preferred_element_type=jnp.float32,
                 precision=jax.lax.Precision.HIGHEST)
    fp = jnp.maximum(fp + bf_ref[...], 0.0)
    fp3 = fp.reshape(R_BLK, K, OUT)
    fmean = jnp.mean(fp3, axis=1, keepdims=True)
    fc = (fp3 - fmean).reshape(R_BLK * K, OUT)
    al = jnp.dot(x, wac_ref[...], preferred_element_type=jnp.float32,
                 precision=jax.lax.Precision.HIGHEST)
    al = al + jnp.dot(fc, waf_ref[...], preferred_element_type=jnp.float32,
                      precision=jax.lax.Precision.HIGHEST)
    al = jnp.maximum(al + ba_ref[...], 0.0)
    a3 = al.reshape(R_BLK, K, OUT)
    amax = jnp.max(a3, axis=1, keepdims=True)
    e = jnp.exp(a3 - amax)
    w = e / jnp.sum(e, axis=1, keepdims=True)
    out_ref[...] = jnp.sum(w * fp3, axis=1)


def _mlp_stage(g, wf, bf, wac, waf, ba):
    rows = g.shape[0] // K
    grid = (rows // R_BLK,)
    return pl.pallas_call(
        _mlp_body,
        grid=grid,
        in_specs=[
            pl.BlockSpec((R_BLK * K, D_PAD), lambda i: (i, 0)),
            pl.BlockSpec((D_PAD, OUT), lambda i: (0, 0)),
            pl.BlockSpec((1, OUT), lambda i: (0, 0)),
            pl.BlockSpec((D_PAD, OUT), lambda i: (0, 0)),
            pl.BlockSpec((OUT, OUT), lambda i: (0, 0)),
            pl.BlockSpec((1, OUT), lambda i: (0, 0)),
        ],
        out_specs=pl.BlockSpec((R_BLK, OUT), lambda i: (i, 0)),
        out_shape=jax.ShapeDtypeStruct((rows, OUT), jnp.float32),
    )(g, wf, bf, wac, waf, ba)


@jax.jit
def kernel(xyz, feat_in, Wf, bf, gf, betaf, Wa, ba, ga, betaa):
    idx_center = jnp.linspace(0.0, N - 1, M).astype(jnp.int32)
    centers = xyz[:, idx_center, :]

    cn2 = jnp.sum(centers * centers, axis=-1)
    xn2 = jnp.sum(xyz * xyz, axis=-1)
    d2 = cn2[:, :, None] + xn2[:, None, :] - 2.0 * jnp.einsum(
        'bmd,bnd->bmn', centers, xyz)
    _, idx_knn = jax.lax.top_k(-d2, K)
    bidx = jnp.arange(B)[:, None, None]
    neigh_xyz = xyz[bidx, idx_knn]
    local_xyz = neigh_xyz - centers[:, :, None, :]
    feat_T = jnp.transpose(feat_in, (0, 2, 1))
    neigh_f = feat_T[bidx, idx_knn]
    cat = jnp.concatenate([local_xyz, neigh_f], axis=-1)
    g = jnp.pad(cat, ((0, 0), (0, 0), (0, 0), (0, D_PAD - D_CAT)))
    g = g.reshape(B * M * K, D_PAD)

    scale_f = gf / jnp.sqrt(1.0 + EPS)
    scale_a = ga / jnp.sqrt(1.0 + EPS)
    wf_eff = jnp.pad(Wf.T, ((0, D_PAD - D_CAT), (0, 0))) * scale_f[None, :]
    bf_eff = (bf * scale_f + betaf)[None, :]
    wa_cat = Wa[:, :D_CAT].T * scale_a[None, :]
    wac_eff = jnp.pad(wa_cat, ((0, D_PAD - D_CAT), (0, 0)))
    waf_eff = Wa[:, D_CAT:].T * scale_a[None, :]
    ba_eff = (ba * scale_a + betaa)[None, :]

    f_region = _mlp_stage(g, wf_eff, bf_eff, wac_eff, waf_eff, ba_eff)
    f_region = f_region.reshape(B, M, OUT).transpose(0, 2, 1)
    return centers, f_region


# speedup vs baseline: 5.1036x; 5.0990x over previous
import dataclasses
import functools

import jax
import jax.numpy as jnp
from jax import lax
from jax.experimental import pallas as pl
from jax.experimental.pallas import tpu as pltpu
from jax.experimental.pallas import tpu_sc as plsc

B, N, C_IN, M, K, OUT = 8, 8192, 64, 2048, 32, 128
D_CAT = 3 + C_IN
D_PAD = 128
F_OFF = 16
EPS = 1e-5

ROWS = B * M
TM = 256
NCHUNK = 64
R_BLK = 128

NW = 32
RPW = ROWS // NW
L = 16


def _dist_body(cb_ref, xb_ref, d2_ref, tub_ref):
    cb = cb_ref[0]
    xb = xb_ref[0]
    cn2 = jnp.sum(cb * cb, axis=0)
    xn2 = jnp.sum(xb * xb, axis=0)
    dot = lax.dot_general(cb.astype(jnp.bfloat16), xb.astype(jnp.bfloat16),
                          (((0,), (0,)), ((), ())),
                          preferred_element_type=jnp.float32)
    d2 = cn2[:, None] + xn2[None, :] - 2.0 * dot
    d2_ref[...] = d2
    cm = jnp.min(d2.reshape(TM, NCHUNK, N // NCHUNK), axis=2)
    for _ in range(K - 1):
        mn = jnp.min(cm, axis=1, keepdims=True)
        cm = jnp.where(cm <= mn, jnp.float32(3.0e38), cm)
    tub_ref[...] = jnp.min(cm, axis=1).reshape(1, 1, TM)


def _dist_stage(centers_t, xyz_t):
    gm = M // TM
    return pl.pallas_call(
        _dist_body,
        grid=(B, gm),
        in_specs=[
            pl.BlockSpec((1, 8, TM), lambda b, i: (b, 0, i)),
            pl.BlockSpec((1, 8, N), lambda b, i: (b, 0, 0)),
        ],
        out_specs=[
            pl.BlockSpec((TM, N), lambda b, i: (b * gm + i, 0)),
            pl.BlockSpec((1, 1, TM), lambda b, i: (b * gm + i, 0, 0)),
        ],
        out_shape=[
            jax.ShapeDtypeStruct((ROWS, N), jnp.float32),
            jax.ShapeDtypeStruct((ROWS // TM, 1, TM), jnp.float32),
        ],
    )(centers_t, xyz_t)


def _sc_compiler_params():
    cp = pltpu.CompilerParams()
    if "needs_layout_passes" in pltpu.CompilerParams.__dataclass_fields__:
        cp = dataclasses.replace(cp, needs_layout_passes=False)
    return cp


def _sortable_u32(v):
    u = plsc.bitcast(v, jnp.uint32)
    neg = v < jnp.full((L,), 0.0, jnp.float32)
    flip = jnp.where(neg, jnp.full((L,), 0xFFFFFFFF, jnp.uint32),
                     jnp.full((L,), 0x80000000, jnp.uint32))
    return u ^ flip


def _make_sc_select_gather():
    mesh = plsc.VectorSubcoreMesh(core_axis_name="c", subcore_axis_name="s")
    i32 = jnp.int32

    @functools.partial(
        pl.kernel,
        mesh=mesh,
        out_type=jax.ShapeDtypeStruct((ROWS * K, D_PAD), jnp.float32),
        scratch_types=[
            pltpu.VMEM((2, N), jnp.float32),
            pltpu.VMEM((RPW, L), jnp.float32),
            pltpu.VMEM((N + L,), i32),
            pltpu.VMEM((N + L,), i32),
            pltpu.VMEM((2, K), i32),
            pltpu.VMEM((2, K, D_PAD), jnp.float32),
            pltpu.SemaphoreType.DMA,
            pltpu.SemaphoreType.DMA,
            pltpu.SemaphoreType.DMA,
            pltpu.SemaphoreType.DMA,
            pltpu.SemaphoreType.DMA,
            pltpu.SemaphoreType.DMA,
            pltpu.SemaphoreType.DMA,
        ],
        compiler_params=_sc_compiler_params(),
    )
    def sc_kernel(d2_hbm, tub_hbm, p2_hbm, out_hbm,
                  d2buf, tubv, ckey, cidx, selv, gbuf,
                  s_in0, s_in1, s_tub, s_g0, s_g1, s_o0, s_o1):
        cid = lax.axis_index("c")
        sid = lax.axis_index("s")
        wid = sid * 2 + cid
        row0 = wid * RPW
        base_n = (row0 // M) * N

        iota = lax.iota(i32, L)
        zero_i = jnp.zeros((L,), i32)
        kvec = jnp.full((L,), K, i32)
        one_u = jnp.full((L,), 1, jnp.uint32)
        ff_u = jnp.full((L,), 0xFFFFFFFF, jnp.uint32)

        pltpu.async_copy(tub_hbm.at[pl.ds(row0, RPW)], tubv, s_tub).wait()

        s_in = (s_in0, s_in1)
        s_g = (s_g0, s_g1)
        s_o = (s_o0, s_o1)

        def start_in(r, p):
            return pltpu.async_copy(d2_hbm.at[r], d2buf.at[p], s_in[p])

        def select_row(r, p):
            ri = r - row0
            tvec = tubv[ri]

            def chunk_body(j, cnt):
                v = d2buf[p, pl.ds(j * L, L)]
                m = v <= tvec
                mi = m.astype(i32)
                pos = cnt + plsc.cumsum(mi) - 1
                plsc.store_scatter(ckey, [pos],
                                   plsc.bitcast(_sortable_u32(v), i32),
                                   mask=m)
                gidx = iota + (base_n + j * L)
                plsc.store_scatter(cidx, [pos], gidx, mask=m)
                return cnt + plsc.all_reduce_population_count(m)

            cnt = lax.fori_loop(0, N // L, chunk_body, zero_i)
            plsc.store_scatter(ckey, [cnt + iota], plsc.bitcast(ff_u, i32))
            c = jnp.max(cnt)
            nv = (c + (L - 1)) // L

            def count_below(t):
                def cb(j, acc):
                    kv = plsc.bitcast(ckey[pl.ds(j * L, L)], jnp.uint32)
                    return acc + plsc.all_reduce_population_count(kv < t)
                return lax.fori_loop(0, nv, cb, zero_i)

            tstar = jnp.zeros((L,), jnp.uint32)
            for i in range(32):
                t2 = tstar | jnp.full((L,), 1 << (31 - i), jnp.uint32)
                nb = count_below(t2)
                tstar = jnp.where(nb < kvec, t2, tstar)
            need_eq = kvec - count_below(tstar)

            def sel_body(j, carry):
                scnt, toteq = carry
                kv = plsc.bitcast(ckey[pl.ds(j * L, L)], jnp.uint32)
                iv = cidx[pl.ds(j * L, L)]
                ltm = kv < tstar
                eqm = kv == tstar
                rank = toteq + plsc.cumsum(eqm.astype(i32))
                msel = ltm | (eqm & (rank <= need_eq))
                pos = scnt + plsc.cumsum(msel.astype(i32)) - 1
                plsc.store_scatter(selv.at[p], [pos], iv, mask=msel)
                return (scnt + plsc.all_reduce_population_count(msel),
                        toteq + plsc.all_reduce_population_count(eqm))

            lax.fori_loop(0, nv, sel_body, (zero_i, zero_i))

        def flush_prev(r, p):
            @pl.when(r - 1 >= row0)
            def _():
                pltpu.make_async_copy(p2_hbm.at[selv.at[1 - p]],
                                      gbuf.at[1 - p], s_g[1 - p]).wait()
                pltpu.async_copy(gbuf.at[1 - p],
                                 out_hbm.at[pl.ds((r - 1) * K, K)], s_o[1 - p])

        def start_gather(r, p):
            @pl.when(r - 2 >= row0)
            def _():
                pltpu.make_async_copy(gbuf.at[p],
                                      out_hbm.at[pl.ds((r - 2) * K, K)],
                                      s_o[p]).wait()
            pltpu.async_copy(p2_hbm.at[selv.at[p]], gbuf.at[p], s_g[p])

        start_in(row0, 0)

        @pl.loop(0, RPW, step=2)
        def _(i):
            r = row0 + i
            start_in(r + 1, 1)
            pltpu.make_async_copy(d2_hbm.at[r], d2buf.at[0], s_in[0]).wait()
            flush_prev(r, 0)
            select_row(r, 0)
            start_gather(r, 0)

            @pl.when(i + 2 < RPW)
            def _():
                start_in(r + 2, 0)
            pltpu.make_async_copy(d2_hbm.at[r + 1], d2buf.at[1],
                                  s_in[1]).wait()
            flush_prev(r + 1, 1)
            select_row(r + 1, 1)
            start_gather(r + 1, 1)

        rlast = row0 + RPW - 1
        pltpu.make_async_copy(p2_hbm.at[selv.at[1]], gbuf.at[1], s_g[1]).wait()
        pltpu.async_copy(gbuf.at[1], out_hbm.at[pl.ds(rlast * K, K)], s_o[1])
        pltpu.make_async_copy(gbuf.at[0], out_hbm.at[pl.ds((rlast - 1) * K, K)],
                              s_o[0]).wait()
        pltpu.make_async_copy(gbuf.at[1], out_hbm.at[pl.ds(rlast * K, K)],
                              s_o[1]).wait()

    return sc_kernel


@functools.cache
def _sc_select_gather():
    return _make_sc_select_gather()


def _mlp_body(g_ref, c_ref, wf_ref, bf_ref, wac_ref, waf_ref, ba_ref, out_ref):
    x = g_ref[...]
    ctr = c_ref[...]
    x = (x.reshape(R_BLK, K, D_PAD) - ctr[:, None, :]).reshape(
        R_BLK * K, D_PAD)
    fp = jnp.dot(x, wf_ref[...], preferred_element_type=jnp.float32,
                 precision=lax.Precision.HIGHEST)
    fp = jnp.maximum(fp + bf_ref[...], 0.0)
    fp3 = fp.reshape(R_BLK, K, OUT)
    fmean = jnp.mean(fp3, axis=1, keepdims=True)
    fc = (fp3 - fmean).reshape(R_BLK * K, OUT)
    al = jnp.dot(x, wac_ref[...], preferred_element_type=jnp.float32,
                 precision=lax.Precision.HIGHEST)
    al = al + jnp.dot(fc, waf_ref[...], preferred_element_type=jnp.float32,
                      precision=lax.Precision.HIGHEST)
    al = jnp.maximum(al + ba_ref[...], 0.0)
    a3 = al.reshape(R_BLK, K, OUT)
    amax = jnp.max(a3, axis=1, keepdims=True)
    e = jnp.exp(a3 - amax)
    w = e / jnp.sum(e, axis=1, keepdims=True)
    out_ref[...] = jnp.sum(w * fp3, axis=1)


def _mlp_stage(g, centers_pad, wf, bf, wac, waf, ba):
    return pl.pallas_call(
        _mlp_body,
        grid=(ROWS // R_BLK,),
        in_specs=[
            pl.BlockSpec((R_BLK * K, D_PAD), lambda i: (i, 0)),
            pl.BlockSpec((R_BLK, D_PAD), lambda i: (i, 0)),
            pl.BlockSpec((D_PAD, OUT), lambda i: (0, 0)),
            pl.BlockSpec((1, OUT), lambda i: (0, 0)),
            pl.BlockSpec((D_PAD, OUT), lambda i: (0, 0)),
            pl.BlockSpec((OUT, OUT), lambda i: (0, 0)),
            pl.BlockSpec((1, OUT), lambda i: (0, 0)),
        ],
        out_specs=pl.BlockSpec((R_BLK, OUT), lambda i: (i, 0)),
        out_shape=jax.ShapeDtypeStruct((ROWS, OUT), jnp.float32),
    )(g, centers_pad, wf, bf, wac, waf, ba)


def _pad_rows(w, off, total):
    return jnp.pad(w, ((off, total - off - w.shape[0]), (0, 0)))


@jax.jit
def kernel(xyz, feat_in, Wf, bf, gf, betaf, Wa, ba, ga, betaa):
    idx_center = jnp.linspace(0.0, N - 1, M).astype(jnp.int32)
    centers = xyz[:, idx_center, :]

    xyz_t = jnp.pad(jnp.transpose(xyz, (0, 2, 1)), ((0, 0), (0, 5), (0, 0)))
    centers_t = jnp.pad(jnp.transpose(centers, (0, 2, 1)),
                        ((0, 0), (0, 5), (0, 0)))
    d2, tub3 = _dist_stage(centers_t, xyz_t)
    tub16 = jnp.broadcast_to(tub3.reshape(ROWS, 1), (ROWS, L))

    feat_t = jnp.transpose(feat_in, (0, 2, 1))
    p2 = jnp.concatenate(
        [xyz, jnp.zeros((B, N, F_OFF - 3), jnp.float32), feat_t,
         jnp.zeros((B, N, D_PAD - F_OFF - C_IN), jnp.float32)],
        axis=2).reshape(B * N, D_PAD)

    g = _sc_select_gather()(d2, tub16, p2)

    scale_f = gf / jnp.sqrt(1.0 + EPS)
    scale_a = ga / jnp.sqrt(1.0 + EPS)
    wf_eff = (_pad_rows(Wf.T[:3], 0, D_PAD)
              + _pad_rows(Wf.T[3:], F_OFF, D_PAD)) * scale_f[None, :]
    bf_eff = (bf * scale_f + betaf)[None, :]
    wac_eff = (_pad_rows(Wa.T[:3], 0, D_PAD)
               + _pad_rows(Wa.T[3:D_CAT], F_OFF, D_PAD)) * scale_a[None, :]
    waf_eff = Wa.T[D_CAT:] * scale_a[None, :]
    ba_eff = (ba * scale_a + betaa)[None, :]

    centers_pad = jnp.pad(centers.reshape(ROWS, 3), ((0, 0), (0, D_PAD - 3)))

    f_region = _mlp_stage(g, centers_pad, wf_eff, bf_eff, wac_eff, waf_eff,
                          ba_eff)
    f_region = f_region.reshape(B, M, OUT).transpose(0, 2, 1)
    return centers, f_region
